# baseline (speedup 1.0000x reference)
import functools

import numpy as np
import jax
import jax.numpy as jnp
from jax import lax
from jax.experimental import pallas as pl
from jax.experimental.pallas import tpu as pltpu
from jax.experimental.pallas import tpu_sc as plsc

B = 4096
NFIELDS = 26
VOCAB = 100000
D = 32
NF1 = NFIELDS + 1
ZDIM = NF1 * NF1


def _sc_gather(table_t, tail_t, idx2):
    info = plsc.get_sparse_core_info()
    nc, ns = info.num_cores, info.num_subcores
    nw = nc * ns
    rows = table_t.shape[0]
    per_w = rows // nw
    ngrp = B // 16
    half0 = 50048
    half1 = VOCAB - half0
    main1 = 49920
    tail_lo = VOCAB - 128
    cut = half0 + main1
    mesh = plsc.VectorSubcoreMesh(core_axis_name="c", subcore_axis_name="s")

    @functools.partial(
        pl.kernel,
        mesh=mesh,
        out_type=jax.ShapeDtypeStruct((rows, B), jnp.float32),
        scratch_types=[
            pltpu.VMEM((half0,), jnp.float32),
            pltpu.VMEM((main1 + 128,), jnp.float32),
            pltpu.VMEM((B,), jnp.int32),
            pltpu.VMEM((B,), jnp.float32),
            pltpu.SemaphoreType.DMA,
            pltpu.SemaphoreType.DMA,
        ],
        compiler_params=pltpu.CompilerParams(needs_layout_passes=False),
    )
    def k(table_hbm, tail_hbm, idx_hbm, out_hbm, buf_a, buf_b, idx_v, out_v,
          sem_a, sem_b):
        wid = lax.axis_index("s") * nc + lax.axis_index("c")
        base = wid * per_w
        iota16 = lax.iota(jnp.int32, 16)

        def start_a(row):
            pltpu.async_copy(
                table_hbm.at[row].at[pl.ds(0, half0)], buf_a, sem_a)

        def start_b(row):
            pltpu.async_copy(
                table_hbm.at[row].at[pl.ds(half0, main1)],
                buf_b.at[pl.ds(0, main1)], sem_b)
            pltpu.async_copy(
                tail_hbm.at[row], buf_b.at[pl.ds(main1, 128)], sem_b)

        def wait_a(row):
            pltpu.make_async_copy(
                table_hbm.at[row].at[pl.ds(0, half0)], buf_a, sem_a).wait()

        def wait_b(row):
            pltpu.make_async_copy(
                table_hbm.at[row].at[pl.ds(half0, main1)],
                buf_b.at[pl.ds(0, main1)], sem_b).wait()
            pltpu.make_async_copy(
                tail_hbm.at[row], buf_b.at[pl.ds(main1, 128)], sem_b).wait()

        def gather_pass(buf, off, size, first):
            def body(t, _):
                o = pl.multiple_of(t * 64, 64)
                for u in range(4):
                    ou = o + u * 16
                    idx16 = idx_v[pl.ds(ou, 16)]
                    local = idx16 - off
                    if not first:
                        local = jnp.where(idx16 >= cut,
                                          idx16 - (tail_lo - main1), local)
                    valid = (idx16 >= off) & (idx16 < off + size)
                    safe = jnp.where(valid, local, 0)
                    g = plsc.load_gather(buf, [safe])
                    if first:
                        out_v[pl.ds(ou, 16)] = g
                    else:
                        plsc.store_scatter(out_v, [iota16 + ou], g, mask=valid)
                return 0

            lax.fori_loop(0, ngrp // 4, body, 0)

        start_a(base)
        def do_row(j, _):
            row = base + j
            f = row // D
            pltpu.sync_copy(idx_hbm.at[f], idx_v)
            start_b(row)
            wait_a(row)
            gather_pass(buf_a, 0, half0, True)

            @pl.when(j < per_w - 1)
            def _():
                start_a(row + 1)

            wait_b(row)
            gather_pass(buf_b, half0, half1, False)
            pltpu.sync_copy(out_v, out_hbm.at[row])
            return 0

        lax.fori_loop(0, per_w, do_row, 0)

    return k(table_t, tail_t, idx2)


def _tc_body(xt_ref, lyt_ref, bw0, bb0, bw1, bb1, bw2, bb2,
             w0x, w0z, tb0, tw1, tb1, tw2, tb2, out_ref):
    f32 = jnp.float32
    x = xt_ref[...]
    h = jnp.maximum(jnp.dot(bw0[...], x, preferred_element_type=f32) + bb0[...], 0.0)
    h = jnp.maximum(jnp.dot(bw1[...], h, preferred_element_type=f32) + bb1[...], 0.0)
    x3 = jnp.maximum(jnp.dot(bw2[...], h, preferred_element_type=f32) + bb2[...], 0.0)

    t2 = jnp.concatenate([x3, lyt_ref[...]], axis=0)
    bb = t2.shape[1]
    t3 = t2.reshape(NF1, D, bb)
    zrows = []
    for i in range(NF1):
        zrows.append(jnp.sum(t3 * t3[i][None], axis=1))
    zt = jnp.concatenate(zrows, axis=0)

    a = jnp.dot(w0x[...], x3, preferred_element_type=f32)
    a = a + jnp.dot(w0z[...], zt, preferred_element_type=f32) + tb0[...]
    a = jnp.maximum(a, 0.0)
    a = jnp.maximum(jnp.dot(tw1[...], a, preferred_element_type=f32) + tb1[...], 0.0)
    o = jnp.dot(tw2[...], a, preferred_element_type=f32) + tb2[...]
    out_ref[...] = 1.0 / (1.0 + jnp.exp(-o))


def _tc_fused(xt, lyt, bw0, bb0, bw1, bb1, bw2, bb2,
              w0x, w0z, tb0, tw1, tb1, tw2, tb2, block_b=512):
    nb = B // block_b

    def full(a):
        return pl.BlockSpec(a.shape, lambda b: (0,) * a.ndim)

    weights = (bw0, bb0, bw1, bb1, bw2, bb2, w0x, w0z, tb0, tw1, tb1, tw2, tb2)
    return pl.pallas_call(
        _tc_body,
        grid=(nb,),
        in_specs=[
            pl.BlockSpec((13, block_b), lambda b: (0, b)),
            pl.BlockSpec((NFIELDS * D, block_b), lambda b: (0, b)),
        ] + [full(w) for w in weights],
        out_specs=pl.BlockSpec((1, block_b), lambda b: (0, b)),
        out_shape=jax.ShapeDtypeStruct((1, B), jnp.float32),
    )(xt, lyt, *weights)


def kernel(dense_x, lS_i, emb_tables, bot_w0, bot_b0, bot_w1, bot_b1,
           bot_w2, bot_b2, top_w0, top_b0, top_w1, top_b1, top_w2, top_b2):
    table_t = jnp.transpose(emb_tables, (0, 2, 1)).reshape(NFIELDS * D, VOCAB)

    tail_t = table_t[:, VOCAB - 128:]
    lyt = _sc_gather(table_t, tail_t, lS_i.astype(jnp.int32))
    xt = dense_x.T

    li, lj = np.tril_indices(NF1, -1)
    sel = np.asarray(li * NF1 + lj)
    w0x = top_w0[:, :D]
    w0z = jnp.zeros((top_w0.shape[0], ZDIM), jnp.float32).at[:, sel].set(top_w0[:, D:])

    def col(b):
        return b.reshape(-1, 1)

    out = _tc_fused(xt, lyt, bot_w0, col(bot_b0), bot_w1, col(bot_b1),
                    bot_w2, col(bot_b2), w0x, w0z, col(top_b0),
                    top_w1, col(top_b1), top_w2, col(top_b2))
    return out.reshape(B, 1)


# speedup vs baseline: 1.0096x; 1.0096x over previous
import functools

import numpy as np
import jax
import jax.numpy as jnp
from jax import lax
from jax.experimental import pallas as pl
from jax.experimental.pallas import tpu as pltpu
from jax.experimental.pallas import tpu_sc as plsc

B = 4096
NFIELDS = 26
VOCAB = 100000
D = 32
NF1 = NFIELDS + 1
ZDIM = NF1 * NF1


def _sc_gather(table_t, tail_t, idx2):
    info = plsc.get_sparse_core_info()
    nc, ns = info.num_cores, info.num_subcores
    nw = nc * ns
    rows = table_t.shape[0]
    per_w = rows // nw
    ngrp = B // 16
    half0 = 50048
    half1 = VOCAB - half0
    main1 = 49920
    mesh = plsc.VectorSubcoreMesh(core_axis_name="c", subcore_axis_name="s")

    @functools.partial(
        pl.kernel,
        mesh=mesh,
        out_type=jax.ShapeDtypeStruct((rows, B), jnp.float32),
        scratch_types=[
            pltpu.VMEM((half0,), jnp.float32),
            pltpu.VMEM((half1,), jnp.float32),
            pltpu.VMEM((32, 128), jnp.float32),
            pltpu.VMEM((B,), jnp.int32),
            pltpu.VMEM((B,), jnp.float32),
            pltpu.SemaphoreType.DMA,
            pltpu.SemaphoreType.DMA,
        ],
        compiler_params=pltpu.CompilerParams(needs_layout_passes=False),
    )
    def k(table_hbm, tail_hbm, idx_hbm, out_hbm, buf_a, buf_b, tail_stage,
          idx_v, out_v, sem_a, sem_b):
        wid = lax.axis_index("s") * nc + lax.axis_index("c")
        base = wid * per_w
        ab = (base // 8) * 8
        iota16 = lax.iota(jnp.int32, 16)

        def start_a(row):
            pltpu.async_copy(
                table_hbm.at[row].at[pl.ds(0, half0)], buf_a, sem_a)

        def start_b(row):
            pltpu.async_copy(
                table_hbm.at[row].at[pl.ds(half0, main1)],
                buf_b.at[pl.ds(0, main1)], sem_b)

        def wait_a(row):
            pltpu.make_async_copy(
                table_hbm.at[row].at[pl.ds(0, half0)], buf_a, sem_a).wait()

        def wait_b(row):
            pltpu.make_async_copy(
                table_hbm.at[row].at[pl.ds(half0, main1)],
                buf_b.at[pl.ds(0, main1)], sem_b).wait()

        def gather_pass(buf, first):
            def body(t, _):
                o = pl.multiple_of(t * 64, 64)
                for u in range(4):
                    ou = o + u * 16
                    idx16 = idx_v[pl.ds(ou, 16)]
                    if first:
                        safe = jnp.minimum(idx16, half0 - 1)
                        out_v[pl.ds(ou, 16)] = plsc.load_gather(buf, [safe])
                    else:
                        local = jnp.maximum(idx16 - half0, 0)
                        g = plsc.load_gather(buf, [local])
                        plsc.store_scatter(out_v, [iota16 + ou], g,
                                           mask=idx16 >= half0)
                return 0

            lax.fori_loop(0, ngrp // 4, body, 0)

        start_a(base)
        pltpu.sync_copy(
            tail_hbm.at[pl.ds(pl.multiple_of(ab, 8), 32)], tail_stage)

        def do_row(j, _):
            row = base + j
            f = row // D
            pltpu.sync_copy(idx_hbm.at[f], idx_v)
            start_b(row)
            wait_a(row)
            gather_pass(buf_a, True)

            @pl.when(j < per_w - 1)
            def _():
                start_a(row + 1)

            wait_b(row)
            rloc = row - ab
            for u in range(2):
                buf_b[pl.ds(main1 + u * 16, 16)] = (
                    tail_stage[rloc, pl.ds(96 + u * 16, 16)])
            gather_pass(buf_b, False)
            pltpu.sync_copy(out_v, out_hbm.at[row])
            return 0

        lax.fori_loop(0, per_w, do_row, 0)

    return k(table_t, tail_t, idx2)


def _tc_body(xt_ref, lyt_ref, bw0, bb0, bw1, bb1, bw2, bb2,
             w0x, w0z, tb0, tw1, tb1, tw2, tb2, out_ref):
    f32 = jnp.float32
    x = xt_ref[...]
    h = jnp.maximum(jnp.dot(bw0[...], x, preferred_element_type=f32) + bb0[...], 0.0)
    h = jnp.maximum(jnp.dot(bw1[...], h, preferred_element_type=f32) + bb1[...], 0.0)
    x3 = jnp.maximum(jnp.dot(bw2[...], h, preferred_element_type=f32) + bb2[...], 0.0)

    t2 = jnp.concatenate([x3, lyt_ref[...]], axis=0)
    bb = t2.shape[1]
    t3 = t2.reshape(NF1, D, bb)
    zrows = []
    for i in range(NF1):
        zrows.append(jnp.sum(t3 * t3[i][None], axis=1))
    zt = jnp.concatenate(zrows, axis=0)

    a = jnp.dot(w0x[...], x3, preferred_element_type=f32)
    a = a + jnp.dot(w0z[...], zt, preferred_element_type=f32) + tb0[...]
    a = jnp.maximum(a, 0.0)
    a = jnp.maximum(jnp.dot(tw1[...], a, preferred_element_type=f32) + tb1[...], 0.0)
    o = jnp.dot(tw2[...], a, preferred_element_type=f32) + tb2[...]
    out_ref[...] = 1.0 / (1.0 + jnp.exp(-o))


def _tc_fused(xt, lyt, bw0, bb0, bw1, bb1, bw2, bb2,
              w0x, w0z, tb0, tw1, tb1, tw2, tb2, block_b=512):
    nb = B // block_b

    def full(a):
        return pl.BlockSpec(a.shape, lambda b: (0,) * a.ndim)

    weights = (bw0, bb0, bw1, bb1, bw2, bb2, w0x, w0z, tb0, tw1, tb1, tw2, tb2)
    return pl.pallas_call(
        _tc_body,
        grid=(nb,),
        in_specs=[
            pl.BlockSpec((13, block_b), lambda b: (0, b)),
            pl.BlockSpec((NFIELDS * D, block_b), lambda b: (0, b)),
        ] + [full(w) for w in weights],
        out_specs=pl.BlockSpec((1, block_b), lambda b: (0, b)),
        out_shape=jax.ShapeDtypeStruct((1, B), jnp.float32),
    )(xt, lyt, *weights)


def kernel(dense_x, lS_i, emb_tables, bot_w0, bot_b0, bot_w1, bot_b1,
           bot_w2, bot_b2, top_w0, top_b0, top_w1, top_b1, top_w2, top_b2):
    table_t = jnp.transpose(emb_tables, (0, 2, 1)).reshape(NFIELDS * D, VOCAB)

    tail_t = table_t[:, VOCAB - 128:]
    lyt = _sc_gather(table_t, tail_t, lS_i.astype(jnp.int32))
    xt = dense_x.T

    li, lj = np.tril_indices(NF1, -1)
    sel = np.asarray(li * NF1 + lj)
    w0x = top_w0[:, :D]
    w0z = jnp.zeros((top_w0.shape[0], ZDIM), jnp.float32).at[:, sel].set(top_w0[:, D:])

    def col(b):
        return b.reshape(-1, 1)

    out = _tc_fused(xt, lyt, bot_w0, col(bot_b0), bot_w1, col(bot_b1),
                    bot_w2, col(bot_b2), w0x, w0z, col(top_b0),
                    top_w1, col(top_b1), top_w2, col(top_b2))
    return out.reshape(B, 1)


# speedup vs baseline: 1.3152x; 1.3026x over previous
import functools

import numpy as np
import jax
import jax.numpy as jnp
from jax import lax
from jax.experimental import pallas as pl
from jax.experimental.pallas import tpu as pltpu
from jax.experimental.pallas import tpu_sc as plsc

B = 4096
NFIELDS = 26
VOCAB = 100000
D = 32
NF1 = NFIELDS + 1
ZDIM = NF1 * NF1


def _sc_gather(table_t, idx2):
    info = plsc.get_sparse_core_info()
    nc, ns = info.num_cores, info.num_subcores
    nw = nc * ns
    rows = table_t.shape[0]
    per_w = rows // nw
    ngrp = B // 16
    mesh = plsc.VectorSubcoreMesh(core_axis_name="c", subcore_axis_name="s")

    @functools.partial(
        pl.kernel,
        mesh=mesh,
        out_type=jax.ShapeDtypeStruct((rows, B), jnp.float32),
        scratch_types=[
            pltpu.VMEM((VOCAB,), jnp.float32),
            pltpu.VMEM((B,), jnp.int32),
            pltpu.VMEM((B,), jnp.float32),
            pltpu.VMEM((B,), jnp.float32),
            pltpu.SemaphoreType.DMA,
            pltpu.SemaphoreType.DMA,
        ],
        compiler_params=pltpu.CompilerParams(needs_layout_passes=False),
    )
    def k(table_hbm, idx_hbm, out_hbm, row_v, idx_v, out0, out1, sem0, sem1):
        wid = lax.axis_index("s") * nc + lax.axis_index("c")
        base = wid * per_w

        def gather_to(out_buf):
            def body(t, _):
                o = pl.multiple_of(t * 64, 64)
                for u in range(4):
                    ou = o + u * 16
                    idx16 = idx_v[pl.ds(ou, 16)]
                    out_buf[pl.ds(ou, 16)] = plsc.load_gather(row_v, [idx16])
                return 0

            lax.fori_loop(0, ngrp // 4, body, 0)

        def one(jj, row, out_buf, sem, prev_f):
            f = row // D

            @pl.when(f != prev_f)
            def _():
                pltpu.sync_copy(idx_hbm.at[f], idx_v)

            pltpu.sync_copy(table_hbm.at[row], row_v)

            @pl.when(jj > 0)
            def _():
                pltpu.make_async_copy(out_buf, out_hbm.at[row], sem).wait()

            gather_to(out_buf)
            pltpu.async_copy(out_buf, out_hbm.at[row], sem)
            return f

        def do_pair(jj, prev_f):
            r0 = base + 2 * jj
            f0 = one(jj, r0, out0, sem0, prev_f)
            return one(jj, r0 + 1, out1, sem1, f0)

        lax.fori_loop(0, per_w // 2, do_pair, jnp.int32(-1))
        last = base + per_w - 2
        pltpu.make_async_copy(out0, out_hbm.at[last], sem0).wait()
        pltpu.make_async_copy(out1, out_hbm.at[last + 1], sem1).wait()

    return k(table_t, idx2)


def _tc_body(xt_ref, lyt_ref, bw0, bb0, bw1, bb1, bw2, bb2,
             w0x, w0z, tb0, tw1, tb1, tw2, tb2, out_ref):
    f32 = jnp.float32
    x = xt_ref[...]
    h = jnp.maximum(jnp.dot(bw0[...], x, preferred_element_type=f32) + bb0[...], 0.0)
    h = jnp.maximum(jnp.dot(bw1[...], h, preferred_element_type=f32) + bb1[...], 0.0)
    x3 = jnp.maximum(jnp.dot(bw2[...], h, preferred_element_type=f32) + bb2[...], 0.0)

    t2 = jnp.concatenate([x3, lyt_ref[...]], axis=0)
    bb = t2.shape[1]
    t3 = t2.reshape(NF1, D, bb)
    zrows = []
    for i in range(NF1):
        zrows.append(jnp.sum(t3 * t3[i][None], axis=1))
    zt = jnp.concatenate(zrows, axis=0)

    a = jnp.dot(w0x[...], x3, preferred_element_type=f32)
    a = a + jnp.dot(w0z[...], zt, preferred_element_type=f32) + tb0[...]
    a = jnp.maximum(a, 0.0)
    a = jnp.maximum(jnp.dot(tw1[...], a, preferred_element_type=f32) + tb1[...], 0.0)
    o = jnp.dot(tw2[...], a, preferred_element_type=f32) + tb2[...]
    out_ref[...] = 1.0 / (1.0 + jnp.exp(-o))


def _tc_fused(xt, lyt, bw0, bb0, bw1, bb1, bw2, bb2,
              w0x, w0z, tb0, tw1, tb1, tw2, tb2, block_b=512):
    nb = B // block_b

    def full(a):
        return pl.BlockSpec(a.shape, lambda b: (0,) * a.ndim)

    weights = (bw0, bb0, bw1, bb1, bw2, bb2, w0x, w0z, tb0, tw1, tb1, tw2, tb2)
    return pl.pallas_call(
        _tc_body,
        grid=(nb,),
        in_specs=[
            pl.BlockSpec((13, block_b), lambda b: (0, b)),
            pl.BlockSpec((NFIELDS * D, block_b), lambda b: (0, b)),
        ] + [full(w) for w in weights],
        out_specs=pl.BlockSpec((1, block_b), lambda b: (0, b)),
        out_shape=jax.ShapeDtypeStruct((1, B), jnp.float32),
    )(xt, lyt, *weights)


def kernel(dense_x, lS_i, emb_tables, bot_w0, bot_b0, bot_w1, bot_b1,
           bot_w2, bot_b2, top_w0, top_b0, top_w1, top_b1, top_w2, top_b2):
    table_t = jnp.transpose(emb_tables, (0, 2, 1)).reshape(NFIELDS * D, VOCAB)

    lyt = _sc_gather(table_t, lS_i.astype(jnp.int32))
    xt = dense_x.T

    li, lj = np.tril_indices(NF1, -1)
    sel = np.asarray(li * NF1 + lj)
    w0x = top_w0[:, :D]
    w0z = jnp.zeros((top_w0.shape[0], ZDIM), jnp.float32).at[:, sel].set(top_w0[:, D:])

    def col(b):
        return b.reshape(-1, 1)

    out = _tc_fused(xt, lyt, bot_w0, col(bot_b0), bot_w1, col(bot_b1),
                    bot_w2, col(bot_b2), w0x, w0z, col(top_b0),
                    top_w1, col(top_b1), top_w2, col(top_b2))
    return out.reshape(B, 1)


# speedup vs baseline: 1.3370x; 1.0166x over previous
import functools

import numpy as np
import jax
import jax.numpy as jnp
from jax import lax
from jax.experimental import pallas as pl
from jax.experimental.pallas import tpu as pltpu
from jax.experimental.pallas import tpu_sc as plsc

B = 4096
NFIELDS = 26
VOCAB = 100000
D = 32
NF1 = NFIELDS + 1
ZDIM = NF1 * NF1


def _sc_gather(table_t, idx2):
    info = plsc.get_sparse_core_info()
    nc, ns = info.num_cores, info.num_subcores
    nw = nc * ns
    rows = table_t.shape[0]
    per_w = rows // nw
    ngrp = B // 16
    mesh = plsc.VectorSubcoreMesh(core_axis_name="c", subcore_axis_name="s")

    @functools.partial(
        pl.kernel,
        mesh=mesh,
        out_type=jax.ShapeDtypeStruct((rows, B), jnp.float32),
        scratch_types=[
            pltpu.VMEM((VOCAB,), jnp.float32),
            pltpu.VMEM((B,), jnp.int32),
            pltpu.VMEM((B,), jnp.float32),
            pltpu.VMEM((B,), jnp.float32),
            pltpu.SemaphoreType.DMA,
            pltpu.SemaphoreType.DMA,
        ],
        compiler_params=pltpu.CompilerParams(needs_layout_passes=False),
    )
    def k(table_hbm, idx_hbm, out_hbm, row_v, idx_v, out0, out1, sem0, sem1):
        wid = lax.axis_index("s") * nc + lax.axis_index("c")
        base = wid * per_w

        def gather_to(out_buf):
            def body(t, _):
                o = pl.multiple_of(t * 64, 64)
                for u in range(4):
                    ou = o + u * 16
                    idx16 = idx_v[pl.ds(ou, 16)]
                    out_buf[pl.ds(ou, 16)] = plsc.load_gather(row_v, [idx16])
                return 0

            lax.fori_loop(0, ngrp // 4, body, 0)

        def one(jj, row, out_buf, sem, prev_f):
            f = row // D

            @pl.when(f != prev_f)
            def _():
                pltpu.sync_copy(idx_hbm.at[f], idx_v)

            pltpu.sync_copy(table_hbm.at[row], row_v)

            @pl.when(jj > 0)
            def _():
                pltpu.make_async_copy(out_buf, out_hbm.at[row], sem).wait()

            gather_to(out_buf)
            pltpu.async_copy(out_buf, out_hbm.at[row], sem)
            return f

        def do_pair(jj, prev_f):
            r0 = base + 2 * jj
            f0 = one(jj, r0, out0, sem0, prev_f)
            return one(jj, r0 + 1, out1, sem1, f0)

        lax.fori_loop(0, per_w // 2, do_pair, jnp.int32(-1))
        last = base + per_w - 2
        pltpu.make_async_copy(out0, out_hbm.at[last], sem0).wait()
        pltpu.make_async_copy(out1, out_hbm.at[last + 1], sem1).wait()

    return k(table_t, idx2)


def _tc_body(xt_ref, lyt_ref, bw0, bb0, bw1, bb1, bw2, bb2,
             w0x, w0z, tb0, tw1, tb1, tw2, tb2, out_ref):
    f32 = jnp.float32
    x = xt_ref[...]
    h = jnp.maximum(jnp.dot(bw0[...], x, preferred_element_type=f32) + bb0[...], 0.0)
    h = jnp.maximum(jnp.dot(bw1[...], h, preferred_element_type=f32) + bb1[...], 0.0)
    x3 = jnp.maximum(jnp.dot(bw2[...], h, preferred_element_type=f32) + bb2[...], 0.0)

    t2 = jnp.concatenate([x3, lyt_ref[...]], axis=0)
    bb = t2.shape[1]
    t3 = t2.reshape(NF1, D, bb)
    zrows = []
    for i in range(NF1):
        zrows.append(jnp.sum(t3 * t3[i][None], axis=1))
    zt = jnp.concatenate(zrows, axis=0)

    a = jnp.dot(w0x[...], x3, preferred_element_type=f32)
    a = a + jnp.dot(w0z[...], zt, preferred_element_type=f32) + tb0[...]
    a = jnp.maximum(a, 0.0)
    a = jnp.maximum(jnp.dot(tw1[...], a, preferred_element_type=f32) + tb1[...], 0.0)
    o = jnp.dot(tw2[...], a, preferred_element_type=f32) + tb2[...]
    out_ref[...] = 1.0 / (1.0 + jnp.exp(-o))


def _tc_fused(xt, lyt, bw0, bb0, bw1, bb1, bw2, bb2,
              w0x, w0z, tb0, tw1, tb1, tw2, tb2, block_b=1024):
    nb = B // block_b

    def full(a):
        return pl.BlockSpec(a.shape, lambda b: (0,) * a.ndim)

    weights = (bw0, bb0, bw1, bb1, bw2, bb2, w0x, w0z, tb0, tw1, tb1, tw2, tb2)
    return pl.pallas_call(
        _tc_body,
        grid=(nb,),
        in_specs=[
            pl.BlockSpec((13, block_b), lambda b: (0, b)),
            pl.BlockSpec((NFIELDS * D, block_b), lambda b: (0, b)),
        ] + [full(w) for w in weights],
        out_specs=pl.BlockSpec((1, block_b), lambda b: (0, b)),
        out_shape=jax.ShapeDtypeStruct((1, B), jnp.float32),
    )(xt, lyt, *weights)


def kernel(dense_x, lS_i, emb_tables, bot_w0, bot_b0, bot_w1, bot_b1,
           bot_w2, bot_b2, top_w0, top_b0, top_w1, top_b1, top_w2, top_b2):
    table_t = jnp.transpose(emb_tables, (0, 2, 1)).reshape(NFIELDS * D, VOCAB)

    lyt = _sc_gather(table_t, lS_i.astype(jnp.int32))
    xt = dense_x.T

    li, lj = np.tril_indices(NF1, -1)
    sel = np.asarray(li * NF1 + lj)
    w0x = top_w0[:, :D]
    w0z = jnp.zeros((top_w0.shape[0], ZDIM), jnp.float32).at[:, sel].set(top_w0[:, D:])

    def col(b):
        return b.reshape(-1, 1)

    out = _tc_fused(xt, lyt, bot_w0, col(bot_b0), bot_w1, col(bot_b1),
                    bot_w2, col(bot_b2), w0x, w0z, col(top_b0),
                    top_w1, col(top_b1), top_w2, col(top_b2))
    return out.reshape(B, 1)


# speedup vs baseline: 1.3761x; 1.0292x over previous
import functools

import numpy as np
import jax
import jax.numpy as jnp
from jax import lax
from jax.experimental import pallas as pl
from jax.experimental.pallas import tpu as pltpu
from jax.experimental.pallas import tpu_sc as plsc

B = 4096
NFIELDS = 26
VOCAB = 100000
D = 32
NF1 = NFIELDS + 1
ZDIM = NF1 * NF1


def _sc_gather(table_t, idx2):
    info = plsc.get_sparse_core_info()
    nc, ns = info.num_cores, info.num_subcores
    nw = nc * ns
    rows = table_t.shape[0]
    per_w = rows // nw
    ngrp = B // 16
    mesh = plsc.VectorSubcoreMesh(core_axis_name="c", subcore_axis_name="s")

    @functools.partial(
        pl.kernel,
        mesh=mesh,
        out_type=jax.ShapeDtypeStruct((rows, B), jnp.float32),
        scratch_types=[
            pltpu.VMEM((VOCAB,), jnp.float32),
            pltpu.VMEM((B,), jnp.int32),
            pltpu.VMEM((B,), jnp.float32),
            pltpu.VMEM((B,), jnp.float32),
            pltpu.SemaphoreType.DMA,
            pltpu.SemaphoreType.DMA,
        ],
        compiler_params=pltpu.CompilerParams(needs_layout_passes=False),
    )
    def k(table_hbm, idx_hbm, out_hbm, row_v, idx_v, out0, out1, sem0, sem1):
        wid = lax.axis_index("s") * nc + lax.axis_index("c")
        base = wid * per_w

        def gather_to(out_buf):
            def body(t, _):
                o = pl.multiple_of(t * 64, 64)
                for u in range(4):
                    ou = o + u * 16
                    idx16 = idx_v[pl.ds(ou, 16)]
                    out_buf[pl.ds(ou, 16)] = plsc.load_gather(row_v, [idx16])
                return 0

            lax.fori_loop(0, ngrp // 4, body, 0)

        def one(jj, row, out_buf, sem, prev_f):
            f = row // D

            @pl.when(f != prev_f)
            def _():
                pltpu.sync_copy(idx_hbm.at[f], idx_v)

            pltpu.sync_copy(table_hbm.at[row], row_v)

            @pl.when(jj > 0)
            def _():
                pltpu.make_async_copy(out_buf, out_hbm.at[row], sem).wait()

            gather_to(out_buf)
            pltpu.async_copy(out_buf, out_hbm.at[row], sem)
            return f

        def do_pair(jj, prev_f):
            r0 = base + 2 * jj
            f0 = one(jj, r0, out0, sem0, prev_f)
            return one(jj, r0 + 1, out1, sem1, f0)

        lax.fori_loop(0, per_w // 2, do_pair, jnp.int32(-1))
        last = base + per_w - 2
        pltpu.make_async_copy(out0, out_hbm.at[last], sem0).wait()
        pltpu.make_async_copy(out1, out_hbm.at[last + 1], sem1).wait()

    return k(table_t, idx2)


def _tc_body(xt_ref, lyt_ref, bw0, bb0, bw1, bb1, bw2, bb2,
             w0x, w0z, tb0, tw1, tb1, tw2, tb2, out_ref):
    f32 = jnp.float32
    x = xt_ref[...]
    h = jnp.maximum(jnp.dot(bw0[...], x, preferred_element_type=f32) + bb0[...], 0.0)
    h = jnp.maximum(jnp.dot(bw1[...], h, preferred_element_type=f32) + bb1[...], 0.0)
    x3 = jnp.maximum(jnp.dot(bw2[...], h, preferred_element_type=f32) + bb2[...], 0.0)

    t2 = jnp.concatenate([x3, lyt_ref[...]], axis=0)
    bb = t2.shape[1]
    t3 = t2.reshape(NF1, D, bb)
    zrows = []
    for i in range(1, NF1):
        zrows.append(jnp.sum(t3[:i] * t3[i][None], axis=1))
    zt = jnp.concatenate(zrows, axis=0)

    a = jnp.dot(w0x[...], x3, preferred_element_type=f32)
    a = a + jnp.dot(w0z[...], zt, preferred_element_type=f32) + tb0[...]
    a = jnp.maximum(a, 0.0)
    a = jnp.maximum(jnp.dot(tw1[...], a, preferred_element_type=f32) + tb1[...], 0.0)
    o = jnp.dot(tw2[...], a, preferred_element_type=f32) + tb2[...]
    out_ref[...] = 1.0 / (1.0 + jnp.exp(-o))


def _tc_fused(xt, lyt, bw0, bb0, bw1, bb1, bw2, bb2,
              w0x, w0z, tb0, tw1, tb1, tw2, tb2, block_b=1024):
    nb = B // block_b

    def full(a):
        return pl.BlockSpec(a.shape, lambda b: (0,) * a.ndim)

    weights = (bw0, bb0, bw1, bb1, bw2, bb2, w0x, w0z, tb0, tw1, tb1, tw2, tb2)
    return pl.pallas_call(
        _tc_body,
        grid=(nb,),
        in_specs=[
            pl.BlockSpec((13, block_b), lambda b: (0, b)),
            pl.BlockSpec((NFIELDS * D, block_b), lambda b: (0, b)),
        ] + [full(w) for w in weights],
        out_specs=pl.BlockSpec((1, block_b), lambda b: (0, b)),
        out_shape=jax.ShapeDtypeStruct((1, B), jnp.float32),
    )(xt, lyt, *weights)


def kernel(dense_x, lS_i, emb_tables, bot_w0, bot_b0, bot_w1, bot_b1,
           bot_w2, bot_b2, top_w0, top_b0, top_w1, top_b1, top_w2, top_b2):
    table_t = jnp.transpose(emb_tables, (0, 2, 1)).reshape(NFIELDS * D, VOCAB)

    lyt = _sc_gather(table_t, lS_i.astype(jnp.int32))
    xt = dense_x.T

    w0x = top_w0[:, :D]
    w0z = top_w0[:, D:]

    def col(b):
        return b.reshape(-1, 1)

    out = _tc_fused(xt, lyt, bot_w0, col(bot_b0), bot_w1, col(bot_b1),
                    bot_w2, col(bot_b2), w0x, w0z, col(top_b0),
                    top_w1, col(top_b1), top_w2, col(top_b2))
    return out.reshape(B, 1)


# speedup vs baseline: 1.3794x; 1.0024x over previous
import functools

import numpy as np
import jax
import jax.numpy as jnp
from jax import lax
from jax.experimental import pallas as pl
from jax.experimental.pallas import tpu as pltpu
from jax.experimental.pallas import tpu_sc as plsc

B = 4096
NFIELDS = 26
VOCAB = 100000
D = 32
NF1 = NFIELDS + 1
ZDIM = NF1 * NF1


def _sc_gather(table_t, idx2):
    info = plsc.get_sparse_core_info()
    nc, ns = info.num_cores, info.num_subcores
    nw = nc * ns
    rows = table_t.shape[0]
    per_w = rows // nw
    ngrp = B // 16
    mesh = plsc.VectorSubcoreMesh(core_axis_name="c", subcore_axis_name="s")

    @functools.partial(
        pl.kernel,
        mesh=mesh,
        out_type=jax.ShapeDtypeStruct((rows, B), jnp.float32),
        scratch_types=[
            pltpu.VMEM((VOCAB,), jnp.float32),
            pltpu.VMEM((B,), jnp.int32),
            pltpu.VMEM((B,), jnp.float32),
            pltpu.VMEM((B,), jnp.float32),
            pltpu.SemaphoreType.DMA,
            pltpu.SemaphoreType.DMA,
        ],
        compiler_params=pltpu.CompilerParams(needs_layout_passes=False),
    )
    def k(table_hbm, idx_hbm, out_hbm, row_v, idx_v, out0, out1, sem0, sem1):
        wid = lax.axis_index("s") * nc + lax.axis_index("c")
        base = wid * per_w

        def gather_to(out_buf):
            def body(t, _):
                o = pl.multiple_of(t * 64, 64)
                for u in range(4):
                    ou = o + u * 16
                    idx16 = idx_v[pl.ds(ou, 16)]
                    out_buf[pl.ds(ou, 16)] = plsc.load_gather(row_v, [idx16])
                return 0

            lax.fori_loop(0, ngrp // 4, body, 0)

        def one(jj, row, out_buf, sem, prev_f):
            f = row // D

            @pl.when(f != prev_f)
            def _():
                pltpu.sync_copy(idx_hbm.at[f], idx_v)

            pltpu.sync_copy(table_hbm.at[row], row_v)

            @pl.when(jj > 0)
            def _():
                pltpu.make_async_copy(out_buf, out_hbm.at[row], sem).wait()

            gather_to(out_buf)
            pltpu.async_copy(out_buf, out_hbm.at[row], sem)
            return f

        def do_pair(jj, prev_f):
            r0 = base + 2 * jj
            f0 = one(jj, r0, out0, sem0, prev_f)
            return one(jj, r0 + 1, out1, sem1, f0)

        lax.fori_loop(0, per_w // 2, do_pair, jnp.int32(-1))
        last = base + per_w - 2
        pltpu.make_async_copy(out0, out_hbm.at[last], sem0).wait()
        pltpu.make_async_copy(out1, out_hbm.at[last + 1], sem1).wait()

    return k(table_t, idx2)


def _tc_body(xt_ref, lyt_ref, bw0, bb0, bw1, bb1, bw2, bb2,
             w0x, w0z, tb0, tw1, tb1, tw2, tb2, out_ref):
    f32 = jnp.float32
    x = xt_ref[...]
    h = jnp.maximum(jnp.dot(bw0[...], x, preferred_element_type=f32) + bb0[...], 0.0)
    h = jnp.maximum(jnp.dot(bw1[...], h, preferred_element_type=f32) + bb1[...], 0.0)
    x3 = jnp.maximum(jnp.dot(bw2[...], h, preferred_element_type=f32) + bb2[...], 0.0)

    bf16 = jnp.bfloat16
    t2 = jnp.concatenate([x3, lyt_ref[...]], axis=0)
    bb = t2.shape[1]
    t3 = t2.reshape(NF1, D, bb).astype(bf16)
    zrows = []
    for i in range(1, NF1):
        zrows.append(jnp.sum(t3[:i] * t3[i][None], axis=1))
    zt = jnp.concatenate(zrows, axis=0)

    a = jnp.dot(w0x[...], x3, preferred_element_type=f32)
    a = a + jnp.dot(w0z[...].astype(bf16), zt, preferred_element_type=f32) + tb0[...]
    a = jnp.maximum(a, 0.0)
    a = jnp.dot(tw1[...].astype(bf16), a.astype(bf16), preferred_element_type=f32)
    a = jnp.maximum(a + tb1[...], 0.0)
    o = jnp.dot(tw2[...], a, preferred_element_type=f32) + tb2[...]
    out_ref[...] = 1.0 / (1.0 + jnp.exp(-o))


def _tc_fused(xt, lyt, bw0, bb0, bw1, bb1, bw2, bb2,
              w0x, w0z, tb0, tw1, tb1, tw2, tb2, block_b=1024):
    nb = B // block_b

    def full(a):
        return pl.BlockSpec(a.shape, lambda b: (0,) * a.ndim)

    weights = (bw0, bb0, bw1, bb1, bw2, bb2, w0x, w0z, tb0, tw1, tb1, tw2, tb2)
    return pl.pallas_call(
        _tc_body,
        grid=(nb,),
        in_specs=[
            pl.BlockSpec((13, block_b), lambda b: (0, b)),
            pl.BlockSpec((NFIELDS * D, block_b), lambda b: (0, b)),
        ] + [full(w) for w in weights],
        out_specs=pl.BlockSpec((1, block_b), lambda b: (0, b)),
        out_shape=jax.ShapeDtypeStruct((1, B), jnp.float32),
    )(xt, lyt, *weights)


def kernel(dense_x, lS_i, emb_tables, bot_w0, bot_b0, bot_w1, bot_b1,
           bot_w2, bot_b2, top_w0, top_b0, top_w1, top_b1, top_w2, top_b2):
    table_t = jnp.transpose(emb_tables, (0, 2, 1)).reshape(NFIELDS * D, VOCAB)

    lyt = _sc_gather(table_t, lS_i.astype(jnp.int32))
    xt = dense_x.T

    w0x = top_w0[:, :D]
    w0z = top_w0[:, D:]

    def col(b):
        return b.reshape(-1, 1)

    out = _tc_fused(xt, lyt, bot_w0, col(bot_b0), bot_w1, col(bot_b1),
                    bot_w2, col(bot_b2), w0x, w0z, col(top_b0),
                    top_w1, col(top_b1), top_w2, col(top_b2))
    return out.reshape(B, 1)
